# baseline (speedup 1.0000x reference)
import functools
import math

import jax
import jax.numpy as jnp
import numpy as np
from jax.experimental import pallas as pl
from jax.experimental.pallas import tpu as pltpu

_FACTOR = 5
_L = 2048
_U = 40
_NEG = -1e9


_U32 = np.uint32


def _tf_rounds(x0, x1, rots):
    for r in rots:
        x0 = (x0 + x1).astype(_U32)
        x1 = ((x1 << _U32(r)) | (x1 >> _U32(32 - r))).astype(_U32)
        x1 = x0 ^ x1
    return x0, x1


def _threefry2x32(k1, k2, x1, x2):
    k1, k2 = _U32(k1), _U32(k2)
    ks = [k1, k2, k1 ^ k2 ^ _U32(0x1BD11BDA)]
    r0, r1 = (13, 15, 26, 6), (17, 29, 16, 24)
    x = [(x1 + ks[0]).astype(_U32), (x2 + ks[1]).astype(_U32)]
    for i, rots in enumerate((r0, r1, r0, r1, r0)):
        x = _tf_rounds(*x, rots)
        a, b = ks[(i + 1) % 3], ks[(i + 2) % 3]
        x = [(x[0] + a).astype(_U32), (x[1] + b + _U32(i + 1)).astype(_U32)]
    return x


def _np_randint(seed, shape, span):
    k1 = _U32(np.uint64(seed) >> np.uint64(32))
    k2 = _U32(np.uint64(seed) & np.uint64(0xFFFFFFFF))
    b1, b2 = _threefry2x32(k1, k2, np.zeros(2, _U32), np.arange(2, dtype=_U32))
    lin = np.arange(int(np.prod(shape)), dtype=np.uint64)
    c1 = (lin >> np.uint64(32)).astype(_U32)
    c2 = (lin & np.uint64(0xFFFFFFFF)).astype(_U32)
    hb1, hb2 = _threefry2x32(b1[0], b2[0], c1, c2)
    lb1, lb2 = _threefry2x32(b1[1], b2[1], c1, c2)
    higher, lower = (hb1 ^ hb2).reshape(shape), (lb1 ^ lb2).reshape(shape)
    span_u = _U32(span)
    mult = _U32((int(2 ** 16) % span) ** 2 % span)
    off = ((higher % span_u) * mult + (lower % span_u)).astype(_U32) % span_u
    return off.astype(np.int64)


def _build_counts_t():
    idx = _np_randint(42, (_L, _U), _L)
    counts_t = np.zeros((_L, _L), dtype=np.int8)
    np.add.at(counts_t, (idx.reshape(-1), np.repeat(np.arange(_L), _U)), 1)
    return counts_t


_COUNTS_T = _build_counts_t()


def _head_kernel(qc_ref, kc_ref, qp_ref, kp_ref, vp_ref, c_ref, o_ref,
                 m2_ref, rank_ref, scat_ref, selc_ref, *, bq1, bq2):
    L = _L
    D = vp_ref.shape[-1]
    scale = 1.0 / math.sqrt(D)
    par = jax.lax.rem(pl.program_id(0), 2)

    K = kp_ref[0]
    V = vp_ref[0]
    m_row = m2_ref[pl.ds(1 - par, 1), :]
    m_col = jnp.transpose(m_row, (1, 0))
    j_iota = jax.lax.broadcasted_iota(jnp.int32, (L, bq1), 0)
    l_iota = jax.lax.broadcasted_iota(jnp.int32, (L, bq1), 1)
    d_iota = j_iota - l_iota

    def s1rk(i, _):
        qb = qc_ref[0, pl.ds(i * bq1, bq1), :]
        st = jax.lax.dot_general(
            kc_ref[0], qb, (((1,), (1,)), ((), ())),
            preferred_element_type=jnp.float32,
        )
        c = c_ref[:, pl.ds(i * bq1, bq1)]
        cf = c.astype(jnp.float32)
        mx = jnp.max(jnp.where(cf > 0.5, st, -3e38), axis=0, keepdims=True)
        sm = jnp.sum(cf * st, axis=0, keepdims=True)
        m2_ref[pl.ds(par, 1), pl.ds(i * bq1, bq1)] = mx - sm * (1.0 / L)

        mb = m2_ref[pl.ds(1 - par, 1), pl.ds(i * bq1, bq1)]
        gt = m_col > mb
        tie = (m_col == mb) & (d_iota < i * bq1)
        cnt = jnp.sum(jnp.where(gt | tie, 1.0, 0.0), axis=0, keepdims=True)
        rank_ref[:, pl.ds(i * bq1, bq1)] = cnt
        return 0

    jax.lax.fori_loop(0, L // bq1, s1rk, 0, unroll=False)

    rank_row = rank_ref[...].astype(jnp.int32)
    t_iota = jax.lax.broadcasted_iota(jnp.int32, (_U, L), 0)
    p40 = jnp.where(rank_row == t_iota, 1.0, 0.0)

    qsel = jax.lax.dot_general(
        p40, qp_ref[0], (((1,), (0,)), ((), ())),
        preferred_element_type=jnp.float32,
    )
    iota_row = jax.lax.broadcasted_iota(jnp.int32, (1, L), 1).astype(jnp.float32)
    qidx = jax.lax.dot_general(
        iota_row, p40, (((1,), (1,)), ((), ())),
        preferred_element_type=jnp.float32,
    ).astype(jnp.int32)
    st = jax.lax.dot_general(
        K, qsel, (((1,), (1,)), ((), ())),
        preferred_element_type=jnp.float32,
    ) * scale
    k_iota = jax.lax.broadcasted_iota(jnp.int32, (L, _U), 0)
    st = jnp.where(k_iota > qidx, _NEG, st)
    st = st - jnp.max(st, axis=0, keepdims=True)
    e = jnp.exp(st)
    p = e / jnp.sum(e, axis=0, keepdims=True)
    osel = jax.lax.dot_general(
        p, V, (((0,), (0,)), ((), ())),
        preferred_element_type=jnp.float32,
    )

    scat_ref[...] = jax.lax.dot_general(
        p40, osel, (((0,), (0,)), ((), ())),
        preferred_element_type=jnp.float32,
    )
    selc_ref[...] = jax.lax.dot_general(
        p40, jnp.ones((_U, 1), jnp.float32), (((0,), (0,)), ((), ())),
        preferred_element_type=jnp.float32,
    )

    r2 = jax.lax.broadcasted_iota(jnp.int32, (bq2, bq2), 0)
    c2 = jax.lax.broadcasted_iota(jnp.int32, (bq2, bq2), 1)
    tri = (r2 >= c2).astype(jnp.float32)

    def s3(i, carry_sum):
        vb = vp_ref[0, pl.ds(i * bq2, bq2), :]
        ctx = jax.lax.dot_general(
            tri, vb, (((1,), (0,)), ((), ())),
            preferred_element_type=jnp.float32,
        ) + carry_sum
        selb = selc_ref[pl.ds(i * bq2, bq2), :]
        scatb = scat_ref[pl.ds(i * bq2, bq2), :]
        o_ref[0, pl.ds(i * bq2, bq2), :] = ctx + selb * (scatb - ctx)
        return carry_sum + jnp.sum(vb, axis=0, keepdims=True)

    jax.lax.fori_loop(0, L // bq2, s3, jnp.zeros((1, D), jnp.float32),
                      unroll=False)


@jax.jit
def kernel(queries, keys, values):
    B, L, H, D = queries.shape
    Q = jnp.transpose(queries, (0, 2, 1, 3)).reshape(B * H, L, D)
    K = jnp.transpose(keys, (0, 2, 1, 3)).reshape(B * H, L, D)
    V = jnp.transpose(values, (0, 2, 1, 3)).reshape(B * H, L, D)
    counts_t = jnp.asarray(_COUNTS_T)

    bq1, bq2 = 512, 256
    nh = B * H
    cur = lambda h: (jnp.minimum(h, nh - 1), 0, 0)
    prev = lambda h: (jnp.maximum(h - 1, 0), 0, 0)
    out = pl.pallas_call(
        functools.partial(_head_kernel, bq1=bq1, bq2=bq2),
        grid=(nh + 1,),
        in_specs=[
            pl.BlockSpec((1, L, D), cur),
            pl.BlockSpec((1, L, D), cur),
            pl.BlockSpec((1, L, D), prev),
            pl.BlockSpec((1, L, D), prev),
            pl.BlockSpec((1, L, D), prev),
            pl.BlockSpec((L, L), lambda h: (0, 0)),
        ],
        out_specs=pl.BlockSpec((1, L, D), prev),
        out_shape=jax.ShapeDtypeStruct((nh, L, D), jnp.float32),
        scratch_shapes=[
            pltpu.VMEM((2, L), jnp.float32),
            pltpu.VMEM((1, L), jnp.float32),
            pltpu.VMEM((L, D), jnp.float32),
            pltpu.VMEM((L, 1), jnp.float32),
        ],
        compiler_params=pltpu.CompilerParams(
            dimension_semantics=("arbitrary",),
            vmem_limit_bytes=100 * 1024 * 1024,
        ),
    )(Q, K, Q, K, V, counts_t)

    return jnp.transpose(out.reshape(B, H, L, D), (0, 2, 1, 3))


# speedup vs baseline: 1.4692x; 1.4692x over previous
import functools
import math

import jax
import jax.numpy as jnp
import numpy as np
from jax.experimental import pallas as pl
from jax.experimental.pallas import tpu as pltpu

_FACTOR = 5
_L = 2048
_U = 40
_NEG = -1e9


_U32 = np.uint32


def _tf_rounds(x0, x1, rots):
    for r in rots:
        x0 = (x0 + x1).astype(_U32)
        x1 = ((x1 << _U32(r)) | (x1 >> _U32(32 - r))).astype(_U32)
        x1 = x0 ^ x1
    return x0, x1


def _threefry2x32(k1, k2, x1, x2):
    k1, k2 = _U32(k1), _U32(k2)
    ks = [k1, k2, k1 ^ k2 ^ _U32(0x1BD11BDA)]
    r0, r1 = (13, 15, 26, 6), (17, 29, 16, 24)
    x = [(x1 + ks[0]).astype(_U32), (x2 + ks[1]).astype(_U32)]
    for i, rots in enumerate((r0, r1, r0, r1, r0)):
        x = _tf_rounds(*x, rots)
        a, b = ks[(i + 1) % 3], ks[(i + 2) % 3]
        x = [(x[0] + a).astype(_U32), (x[1] + b + _U32(i + 1)).astype(_U32)]
    return x


def _np_randint(seed, shape, span):
    k1 = _U32(np.uint64(seed) >> np.uint64(32))
    k2 = _U32(np.uint64(seed) & np.uint64(0xFFFFFFFF))
    b1, b2 = _threefry2x32(k1, k2, np.zeros(2, _U32), np.arange(2, dtype=_U32))
    lin = np.arange(int(np.prod(shape)), dtype=np.uint64)
    c1 = (lin >> np.uint64(32)).astype(_U32)
    c2 = (lin & np.uint64(0xFFFFFFFF)).astype(_U32)
    hb1, hb2 = _threefry2x32(b1[0], b2[0], c1, c2)
    lb1, lb2 = _threefry2x32(b1[1], b2[1], c1, c2)
    higher, lower = (hb1 ^ hb2).reshape(shape), (lb1 ^ lb2).reshape(shape)
    span_u = _U32(span)
    mult = _U32((int(2 ** 16) % span) ** 2 % span)
    off = ((higher % span_u) * mult + (lower % span_u)).astype(_U32) % span_u
    return off.astype(np.int64)


def _build_counts_t():
    idx = _np_randint(42, (_L, _U), _L)
    counts_t = np.zeros((_L, _L), dtype=np.int8)
    np.add.at(counts_t, (idx.reshape(-1), np.repeat(np.arange(_L), _U)), 1)
    return counts_t


_COUNTS_T = _build_counts_t()


def _head_kernel(qc_ref, kc_ref, qp_ref, kp_ref, vp_ref, c_ref, o_ref,
                 m2_ref, rank_ref, scat_ref, selc_ref, *, bq1, bq2):
    L = _L
    D = vp_ref.shape[-1]
    scale = 1.0 / math.sqrt(D)
    par = jax.lax.rem(pl.program_id(0), 2)

    K = kp_ref[0]
    V = vp_ref[0]
    m_row = m2_ref[pl.ds(1 - par, 1), :]
    m_col = jnp.transpose(m_row, (1, 0))
    j_iota = jax.lax.broadcasted_iota(jnp.int32, (L, bq1), 0)
    l_iota = jax.lax.broadcasted_iota(jnp.int32, (L, bq1), 1)
    d_iota = j_iota - l_iota

    for i in range(L // bq1):
        qb = qc_ref[0, pl.ds(i * bq1, bq1), :]
        st = jax.lax.dot_general(
            kc_ref[0], qb, (((1,), (1,)), ((), ())),
            preferred_element_type=jnp.float32,
        )
        c = c_ref[:, pl.ds(i * bq1, bq1)]
        cf = c.astype(jnp.float32)
        mx = jnp.max(jnp.where(cf > 0.5, st, -3e38), axis=0, keepdims=True)
        sm = jnp.sum(cf * st, axis=0, keepdims=True)
        m2_ref[pl.ds(par, 1), pl.ds(i * bq1, bq1)] = mx - sm * (1.0 / L)

        mb = jax.lax.slice(m_row, (0, i * bq1), (1, (i + 1) * bq1))
        gt = m_col > mb
        tie = (m_col == mb) & (d_iota < i * bq1)
        cnt = jnp.sum(jnp.where(gt | tie, 1.0, 0.0), axis=0, keepdims=True)
        rank_ref[:, pl.ds(i * bq1, bq1)] = cnt

    rank_row = rank_ref[...].astype(jnp.int32)
    t_iota = jax.lax.broadcasted_iota(jnp.int32, (_U, L), 0)
    p40 = jnp.where(rank_row == t_iota, 1.0, 0.0)

    qsel = jax.lax.dot_general(
        p40, qp_ref[0], (((1,), (0,)), ((), ())),
        preferred_element_type=jnp.float32,
    )
    iota_row = jax.lax.broadcasted_iota(jnp.int32, (1, L), 1).astype(jnp.float32)
    qidx = jax.lax.dot_general(
        iota_row, p40, (((1,), (1,)), ((), ())),
        preferred_element_type=jnp.float32,
    ).astype(jnp.int32)
    st = jax.lax.dot_general(
        K, qsel, (((1,), (1,)), ((), ())),
        preferred_element_type=jnp.float32,
    ) * scale
    k_iota = jax.lax.broadcasted_iota(jnp.int32, (L, _U), 0)
    st = jnp.where(k_iota > qidx, _NEG, st)
    st = st - jnp.max(st, axis=0, keepdims=True)
    e = jnp.exp(st)
    p = e / jnp.sum(e, axis=0, keepdims=True)
    osel = jax.lax.dot_general(
        p, V, (((0,), (0,)), ((), ())),
        preferred_element_type=jnp.float32,
    )

    scat_ref[...] = jax.lax.dot_general(
        p40, osel, (((0,), (0,)), ((), ())),
        preferred_element_type=jnp.float32,
    )
    selc_ref[...] = jax.lax.dot_general(
        p40, jnp.ones((_U, 1), jnp.float32), (((0,), (0,)), ((), ())),
        preferred_element_type=jnp.float32,
    )

    r2 = jax.lax.broadcasted_iota(jnp.int32, (bq2, bq2), 0)
    c2 = jax.lax.broadcasted_iota(jnp.int32, (bq2, bq2), 1)
    tri = (r2 >= c2).astype(jnp.float32)

    def s3(i, carry_sum):
        vb = vp_ref[0, pl.ds(i * bq2, bq2), :]
        ctx = jax.lax.dot_general(
            tri, vb, (((1,), (0,)), ((), ())),
            preferred_element_type=jnp.float32,
        ) + carry_sum
        selb = selc_ref[pl.ds(i * bq2, bq2), :]
        scatb = scat_ref[pl.ds(i * bq2, bq2), :]
        o_ref[0, pl.ds(i * bq2, bq2), :] = ctx + selb * (scatb - ctx)
        return carry_sum + jnp.sum(vb, axis=0, keepdims=True)

    jax.lax.fori_loop(0, L // bq2, s3, jnp.zeros((1, D), jnp.float32),
                      unroll=False)


@jax.jit
def kernel(queries, keys, values):
    B, L, H, D = queries.shape
    Q = jnp.transpose(queries, (0, 2, 1, 3)).reshape(B * H, L, D)
    K = jnp.transpose(keys, (0, 2, 1, 3)).reshape(B * H, L, D)
    V = jnp.transpose(values, (0, 2, 1, 3)).reshape(B * H, L, D)
    counts_t = jnp.asarray(_COUNTS_T)

    bq1, bq2 = 512, 256
    nh = B * H
    cur = lambda h: (jnp.minimum(h, nh - 1), 0, 0)
    prev = lambda h: (jnp.maximum(h - 1, 0), 0, 0)
    out = pl.pallas_call(
        functools.partial(_head_kernel, bq1=bq1, bq2=bq2),
        grid=(nh + 1,),
        in_specs=[
            pl.BlockSpec((1, L, D), cur),
            pl.BlockSpec((1, L, D), cur),
            pl.BlockSpec((1, L, D), prev),
            pl.BlockSpec((1, L, D), prev),
            pl.BlockSpec((1, L, D), prev),
            pl.BlockSpec((L, L), lambda h: (0, 0)),
        ],
        out_specs=pl.BlockSpec((1, L, D), prev),
        out_shape=jax.ShapeDtypeStruct((nh, L, D), jnp.float32),
        scratch_shapes=[
            pltpu.VMEM((2, L), jnp.float32),
            pltpu.VMEM((1, L), jnp.float32),
            pltpu.VMEM((L, D), jnp.float32),
            pltpu.VMEM((L, 1), jnp.float32),
        ],
        compiler_params=pltpu.CompilerParams(
            dimension_semantics=("arbitrary",),
            vmem_limit_bytes=100 * 1024 * 1024,
        ),
    )(Q, K, Q, K, V, counts_t)

    return jnp.transpose(out.reshape(B, H, L, D), (0, 2, 1, 3))


# speedup vs baseline: 1.6211x; 1.1034x over previous
import functools
import math

import jax
import jax.numpy as jnp
import numpy as np
from jax.experimental import pallas as pl
from jax.experimental.pallas import tpu as pltpu

_FACTOR = 5
_L = 2048
_U = 40
_NEG = -1e9


_U32 = np.uint32


def _tf_rounds(x0, x1, rots):
    for r in rots:
        x0 = (x0 + x1).astype(_U32)
        x1 = ((x1 << _U32(r)) | (x1 >> _U32(32 - r))).astype(_U32)
        x1 = x0 ^ x1
    return x0, x1


def _threefry2x32(k1, k2, x1, x2):
    k1, k2 = _U32(k1), _U32(k2)
    ks = [k1, k2, k1 ^ k2 ^ _U32(0x1BD11BDA)]
    r0, r1 = (13, 15, 26, 6), (17, 29, 16, 24)
    x = [(x1 + ks[0]).astype(_U32), (x2 + ks[1]).astype(_U32)]
    for i, rots in enumerate((r0, r1, r0, r1, r0)):
        x = _tf_rounds(*x, rots)
        a, b = ks[(i + 1) % 3], ks[(i + 2) % 3]
        x = [(x[0] + a).astype(_U32), (x[1] + b + _U32(i + 1)).astype(_U32)]
    return x


def _np_randint(seed, shape, span):
    k1 = _U32(np.uint64(seed) >> np.uint64(32))
    k2 = _U32(np.uint64(seed) & np.uint64(0xFFFFFFFF))
    b1, b2 = _threefry2x32(k1, k2, np.zeros(2, _U32), np.arange(2, dtype=_U32))
    lin = np.arange(int(np.prod(shape)), dtype=np.uint64)
    c1 = (lin >> np.uint64(32)).astype(_U32)
    c2 = (lin & np.uint64(0xFFFFFFFF)).astype(_U32)
    hb1, hb2 = _threefry2x32(b1[0], b2[0], c1, c2)
    lb1, lb2 = _threefry2x32(b1[1], b2[1], c1, c2)
    higher, lower = (hb1 ^ hb2).reshape(shape), (lb1 ^ lb2).reshape(shape)
    span_u = _U32(span)
    mult = _U32((int(2 ** 16) % span) ** 2 % span)
    off = ((higher % span_u) * mult + (lower % span_u)).astype(_U32) % span_u
    return off.astype(np.int64)


def _build_counts_t():
    idx = _np_randint(42, (_L, _U), _L)
    counts_t = np.zeros((_L, _L), dtype=np.int8)
    np.add.at(counts_t, (idx.reshape(-1), np.repeat(np.arange(_L), _U)), 1)
    return counts_t


_COUNTS_T = _build_counts_t()


def _head_kernel(qc_ref, kc_ref, qp_ref, kp_ref, vp_ref, c_ref, o_ref,
                 m2_ref, rank_ref, *, bq1, bq2):
    L = _L
    D = vp_ref.shape[-1]
    scale = 1.0 / math.sqrt(D)
    par = jax.lax.rem(pl.program_id(0), 2)

    m_row = m2_ref[pl.ds(1 - par, 1), :]
    m_col = jnp.transpose(m_row, (1, 0))
    j_iota = jax.lax.broadcasted_iota(jnp.int32, (L, bq1), 0)
    l_iota = jax.lax.broadcasted_iota(jnp.int32, (L, bq1), 1)
    d_iota = j_iota - l_iota

    for i in range(L // bq1):
        qb = qc_ref[0, pl.ds(i * bq1, bq1), :]
        st = jax.lax.dot_general(
            kc_ref[0], qb, (((1,), (1,)), ((), ())),
            preferred_element_type=jnp.float32,
        )
        c = c_ref[:, pl.ds(i * bq1, bq1)]
        cf = c.astype(jnp.float32)
        mx = jnp.max(jnp.where(cf > 0.5, st, -3e38), axis=0, keepdims=True)
        sm = jnp.sum(cf * st, axis=0, keepdims=True)
        m2_ref[pl.ds(par, 1), pl.ds(i * bq1, bq1)] = mx - sm * (1.0 / L)

        mb = jax.lax.slice(m_row, (0, i * bq1), (1, (i + 1) * bq1))
        gt = m_col > mb
        tie = (m_col == mb) & (d_iota < i * bq1)
        cnt = jnp.sum(jnp.where(gt | tie, 1.0, 0.0), axis=0, keepdims=True)
        rank_ref[:, pl.ds(i * bq1, bq1)] = cnt

    rank_row = rank_ref[...].astype(jnp.int32)
    t_iota = jax.lax.broadcasted_iota(jnp.int32, (_U, L), 0)
    p40 = jnp.where(rank_row == t_iota, 1.0, 0.0)

    qsel = jax.lax.dot_general(
        p40, qp_ref[0], (((1,), (0,)), ((), ())),
        preferred_element_type=jnp.float32,
    )
    iota_row = jax.lax.broadcasted_iota(jnp.int32, (1, L), 1).astype(jnp.float32)
    qidx = jax.lax.dot_general(
        iota_row, p40, (((1,), (1,)), ((), ())),
        preferred_element_type=jnp.float32,
    ).astype(jnp.int32)
    st = jax.lax.dot_general(
        kp_ref[0], qsel, (((1,), (1,)), ((), ())),
        preferred_element_type=jnp.float32,
    ) * scale
    k_iota = jax.lax.broadcasted_iota(jnp.int32, (L, _U), 0)
    st = jnp.where(k_iota > qidx, _NEG, st)
    st = st - jnp.max(st, axis=0, keepdims=True)
    e = jnp.exp(st)
    p = e / jnp.sum(e, axis=0, keepdims=True)
    osel = jax.lax.dot_general(
        p, vp_ref[0], (((0,), (0,)), ((), ())),
        preferred_element_type=jnp.float32,
    )

    scat = jax.lax.dot_general(
        p40, osel, (((0,), (0,)), ((), ())),
        preferred_element_type=jnp.float32,
    )
    selc = jax.lax.dot_general(
        p40, jnp.ones((_U, 1), jnp.float32), (((0,), (0,)), ((), ())),
        preferred_element_type=jnp.float32,
    )

    r2 = jax.lax.broadcasted_iota(jnp.int32, (bq2, bq2), 0)
    c2 = jax.lax.broadcasted_iota(jnp.int32, (bq2, bq2), 1)
    tri = (r2 >= c2).astype(jnp.float32)

    nb = L // bq2
    vbs = [vp_ref[0, pl.ds(i * bq2, bq2), :] for i in range(nb)]
    bsums = [jnp.sum(vb, axis=0, keepdims=True) for vb in vbs]
    carry = jnp.zeros((1, D), jnp.float32)
    for i in range(nb):
        ctx = jax.lax.dot_general(
            tri, vbs[i], (((1,), (0,)), ((), ())),
            preferred_element_type=jnp.float32,
        ) + carry
        selb = jax.lax.slice(selc, (i * bq2, 0), ((i + 1) * bq2, 1))
        scatb = jax.lax.slice(scat, (i * bq2, 0), ((i + 1) * bq2, D))
        o_ref[0, pl.ds(i * bq2, bq2), :] = ctx + selb * (scatb - ctx)
        carry = carry + bsums[i]


@jax.jit
def kernel(queries, keys, values):
    B, L, H, D = queries.shape
    Q = jnp.transpose(queries, (0, 2, 1, 3)).reshape(B * H, L, D)
    K = jnp.transpose(keys, (0, 2, 1, 3)).reshape(B * H, L, D)
    V = jnp.transpose(values, (0, 2, 1, 3)).reshape(B * H, L, D)
    counts_t = jnp.asarray(_COUNTS_T)

    bq1, bq2 = 512, 256
    nh = B * H
    cur = lambda h: (jnp.minimum(h, nh - 1), 0, 0)
    prev = lambda h: (jnp.maximum(h - 1, 0), 0, 0)
    out = pl.pallas_call(
        functools.partial(_head_kernel, bq1=bq1, bq2=bq2),
        grid=(nh + 1,),
        in_specs=[
            pl.BlockSpec((1, L, D), cur),
            pl.BlockSpec((1, L, D), cur),
            pl.BlockSpec((1, L, D), prev),
            pl.BlockSpec((1, L, D), prev),
            pl.BlockSpec((1, L, D), prev),
            pl.BlockSpec((L, L), lambda h: (0, 0)),
        ],
        out_specs=pl.BlockSpec((1, L, D), prev),
        out_shape=jax.ShapeDtypeStruct((nh, L, D), jnp.float32),
        scratch_shapes=[
            pltpu.VMEM((2, L), jnp.float32),
            pltpu.VMEM((1, L), jnp.float32),
        ],
        compiler_params=pltpu.CompilerParams(
            dimension_semantics=("arbitrary",),
            vmem_limit_bytes=100 * 1024 * 1024,
        ),
    )(Q, K, Q, K, V, counts_t)

    return jnp.transpose(out.reshape(B, H, L, D), (0, 2, 1, 3))


# speedup vs baseline: 1.8476x; 1.1397x over previous
import functools
import math

import jax
import jax.numpy as jnp
import numpy as np
from jax.experimental import pallas as pl
from jax.experimental.pallas import tpu as pltpu

_FACTOR = 5
_L = 2048
_U = 40
_NEG = -1e9


_U32 = np.uint32


def _tf_rounds(x0, x1, rots):
    for r in rots:
        x0 = (x0 + x1).astype(_U32)
        x1 = ((x1 << _U32(r)) | (x1 >> _U32(32 - r))).astype(_U32)
        x1 = x0 ^ x1
    return x0, x1


def _threefry2x32(k1, k2, x1, x2):
    k1, k2 = _U32(k1), _U32(k2)
    ks = [k1, k2, k1 ^ k2 ^ _U32(0x1BD11BDA)]
    r0, r1 = (13, 15, 26, 6), (17, 29, 16, 24)
    x = [(x1 + ks[0]).astype(_U32), (x2 + ks[1]).astype(_U32)]
    for i, rots in enumerate((r0, r1, r0, r1, r0)):
        x = _tf_rounds(*x, rots)
        a, b = ks[(i + 1) % 3], ks[(i + 2) % 3]
        x = [(x[0] + a).astype(_U32), (x[1] + b + _U32(i + 1)).astype(_U32)]
    return x


def _np_randint(seed, shape, span):
    k1 = _U32(np.uint64(seed) >> np.uint64(32))
    k2 = _U32(np.uint64(seed) & np.uint64(0xFFFFFFFF))
    b1, b2 = _threefry2x32(k1, k2, np.zeros(2, _U32), np.arange(2, dtype=_U32))
    lin = np.arange(int(np.prod(shape)), dtype=np.uint64)
    c1 = (lin >> np.uint64(32)).astype(_U32)
    c2 = (lin & np.uint64(0xFFFFFFFF)).astype(_U32)
    hb1, hb2 = _threefry2x32(b1[0], b2[0], c1, c2)
    lb1, lb2 = _threefry2x32(b1[1], b2[1], c1, c2)
    higher, lower = (hb1 ^ hb2).reshape(shape), (lb1 ^ lb2).reshape(shape)
    span_u = _U32(span)
    mult = _U32((int(2 ** 16) % span) ** 2 % span)
    off = ((higher % span_u) * mult + (lower % span_u)).astype(_U32) % span_u
    return off.astype(np.int64)


def _build_counts_t():
    idx = _np_randint(42, (_L, _U), _L)
    counts_t = np.zeros((_L, _L), dtype=np.int8)
    np.add.at(counts_t, (idx.reshape(-1), np.repeat(np.arange(_L), _U)), 1)
    return counts_t


_COUNTS_T = _build_counts_t()


def _pair_kernel(qc_ref, kc_ref, qp_ref, kp_ref, vp_ref, c_ref, o_ref,
                 m2_ref, rank_ref, *, bq1, bq2, d):
    L = _L
    scale = 1.0 / math.sqrt(d)
    par = jax.lax.rem(pl.program_id(0), 2)

    j_iota = jax.lax.broadcasted_iota(jnp.int32, (L, bq1), 0)
    l_iota = jax.lax.broadcasted_iota(jnp.int32, (L, bq1), 1)
    d_iota = j_iota - l_iota

    m_rows = []
    m_cols = []
    for hh in range(2):
        m_row = m2_ref[pl.ds(2 * (1 - par) + hh, 1), :]
        m_rows.append(m_row)
        m_cols.append(jnp.transpose(m_row, (1, 0)))

    for i in range(L // bq1):
        c = c_ref[:, pl.ds(i * bq1, bq1)]
        cf = c.astype(jnp.float32)
        for hh in range(2):
            qb = qc_ref[pl.ds(i * bq1, bq1), pl.ds(hh * d, d)]
            kcur = kc_ref[:, pl.ds(hh * d, d)]
            st = jax.lax.dot_general(
                kcur, qb, (((1,), (1,)), ((), ())),
                preferred_element_type=jnp.float32,
            )
            mx = jnp.max(jnp.where(cf > 0.5, st, -3e38), axis=0,
                         keepdims=True)
            sm = jnp.sum(cf * st, axis=0, keepdims=True)
            m2_ref[pl.ds(2 * par + hh, 1), pl.ds(i * bq1, bq1)] = (
                mx - sm * (1.0 / L))

            mb = jax.lax.slice(m_rows[hh], (0, i * bq1), (1, (i + 1) * bq1))
            gt = m_cols[hh] > mb
            tie = (m_cols[hh] == mb) & (d_iota < i * bq1)
            cnt = jnp.sum(jnp.where(gt | tie, 1.0, 0.0), axis=0,
                          keepdims=True)
            rank_ref[pl.ds(hh, 1), pl.ds(i * bq1, bq1)] = cnt

    t_iota = jax.lax.broadcasted_iota(jnp.int32, (_U, L), 0)
    iota_row = jax.lax.broadcasted_iota(jnp.int32, (1, L), 1).astype(
        jnp.float32)
    k_iota = jax.lax.broadcasted_iota(jnp.int32, (L, _U), 0)
    r2 = jax.lax.broadcasted_iota(jnp.int32, (bq2, bq2), 0)
    c2 = jax.lax.broadcasted_iota(jnp.int32, (bq2, bq2), 1)
    tri = (r2 >= c2).astype(jnp.float32)
    nb = L // bq2

    for hh in range(2):
        hs = pl.ds(hh * d, d)
        rank_row = rank_ref[pl.ds(hh, 1), :].astype(jnp.int32)
        p40 = jnp.where(rank_row == t_iota, 1.0, 0.0)

        qsel = jax.lax.dot_general(
            p40, qp_ref[:, hs], (((1,), (0,)), ((), ())),
            preferred_element_type=jnp.float32,
        )
        qidx = jax.lax.dot_general(
            iota_row, p40, (((1,), (1,)), ((), ())),
            preferred_element_type=jnp.float32,
        ).astype(jnp.int32)
        st = jax.lax.dot_general(
            kp_ref[:, hs], qsel, (((1,), (1,)), ((), ())),
            preferred_element_type=jnp.float32,
        ) * scale
        st = jnp.where(k_iota > qidx, _NEG, st)
        st = st - jnp.max(st, axis=0, keepdims=True)
        e = jnp.exp(st)
        p = e / jnp.sum(e, axis=0, keepdims=True)
        osel = jax.lax.dot_general(
            p, vp_ref[:, hs], (((0,), (0,)), ((), ())),
            preferred_element_type=jnp.float32,
        )

        scat = jax.lax.dot_general(
            p40, osel, (((0,), (0,)), ((), ())),
            preferred_element_type=jnp.float32,
        )
        selc = jax.lax.dot_general(
            p40, jnp.ones((_U, 1), jnp.float32), (((0,), (0,)), ((), ())),
            preferred_element_type=jnp.float32,
        )

        vbs = [vp_ref[pl.ds(i * bq2, bq2), hs] for i in range(nb)]
        bsums = [jnp.sum(vb, axis=0, keepdims=True) for vb in vbs]
        carry = jnp.zeros((1, d), jnp.float32)
        for i in range(nb):
            ctx = jax.lax.dot_general(
                tri, vbs[i], (((1,), (0,)), ((), ())),
                preferred_element_type=jnp.float32,
            ) + carry
            selb = jax.lax.slice(selc, (i * bq2, 0), ((i + 1) * bq2, 1))
            scatb = jax.lax.slice(scat, (i * bq2, 0), ((i + 1) * bq2, d))
            o_ref[pl.ds(i * bq2, bq2), hs] = ctx + selb * (scatb - ctx)
            carry = carry + bsums[i]


@jax.jit
def kernel(queries, keys, values):
    B, L, H, D = queries.shape
    HD = H * D
    Q = queries.reshape(B * L, HD)
    K = keys.reshape(B * L, HD)
    V = values.reshape(B * L, HD)
    counts_t = jnp.asarray(_COUNTS_T)

    bq1, bq2 = 512, 256
    npair = H // 2
    cur = lambda g: (0, jnp.minimum(g, npair - 1))
    prev = lambda g: (0, jnp.maximum(g - 1, 0))
    out = pl.pallas_call(
        functools.partial(_pair_kernel, bq1=bq1, bq2=bq2, d=D),
        grid=(npair + 1,),
        in_specs=[
            pl.BlockSpec((L, 2 * D), cur),
            pl.BlockSpec((L, 2 * D), cur),
            pl.BlockSpec((L, 2 * D), prev),
            pl.BlockSpec((L, 2 * D), prev),
            pl.BlockSpec((L, 2 * D), prev),
            pl.BlockSpec((L, L), lambda g: (0, 0)),
        ],
        out_specs=pl.BlockSpec((L, 2 * D), prev),
        out_shape=jax.ShapeDtypeStruct((B * L, HD), jnp.float32),
        scratch_shapes=[
            pltpu.VMEM((4, L), jnp.float32),
            pltpu.VMEM((2, L), jnp.float32),
        ],
        compiler_params=pltpu.CompilerParams(
            dimension_semantics=("arbitrary",),
            vmem_limit_bytes=100 * 1024 * 1024,
        ),
    )(Q, K, Q, K, V, counts_t)

    return out.reshape(B, L, H, D)


# speedup vs baseline: 1.8761x; 1.0154x over previous
import functools
import math

import jax
import jax.numpy as jnp
import numpy as np
from jax.experimental import pallas as pl
from jax.experimental.pallas import tpu as pltpu

_FACTOR = 5
_L = 2048
_U = 40
_NEG = -1e9


_U32 = np.uint32


def _tf_rounds(x0, x1, rots):
    for r in rots:
        x0 = (x0 + x1).astype(_U32)
        x1 = ((x1 << _U32(r)) | (x1 >> _U32(32 - r))).astype(_U32)
        x1 = x0 ^ x1
    return x0, x1


def _threefry2x32(k1, k2, x1, x2):
    k1, k2 = _U32(k1), _U32(k2)
    ks = [k1, k2, k1 ^ k2 ^ _U32(0x1BD11BDA)]
    r0, r1 = (13, 15, 26, 6), (17, 29, 16, 24)
    x = [(x1 + ks[0]).astype(_U32), (x2 + ks[1]).astype(_U32)]
    for i, rots in enumerate((r0, r1, r0, r1, r0)):
        x = _tf_rounds(*x, rots)
        a, b = ks[(i + 1) % 3], ks[(i + 2) % 3]
        x = [(x[0] + a).astype(_U32), (x[1] + b + _U32(i + 1)).astype(_U32)]
    return x


def _np_randint(seed, shape, span):
    k1 = _U32(np.uint64(seed) >> np.uint64(32))
    k2 = _U32(np.uint64(seed) & np.uint64(0xFFFFFFFF))
    b1, b2 = _threefry2x32(k1, k2, np.zeros(2, _U32), np.arange(2, dtype=_U32))
    lin = np.arange(int(np.prod(shape)), dtype=np.uint64)
    c1 = (lin >> np.uint64(32)).astype(_U32)
    c2 = (lin & np.uint64(0xFFFFFFFF)).astype(_U32)
    hb1, hb2 = _threefry2x32(b1[0], b2[0], c1, c2)
    lb1, lb2 = _threefry2x32(b1[1], b2[1], c1, c2)
    higher, lower = (hb1 ^ hb2).reshape(shape), (lb1 ^ lb2).reshape(shape)
    span_u = _U32(span)
    mult = _U32((int(2 ** 16) % span) ** 2 % span)
    off = ((higher % span_u) * mult + (lower % span_u)).astype(_U32) % span_u
    return off.astype(np.int64)


def _build_counts_t():
    idx = _np_randint(42, (_L, _U), _L)
    counts_t = np.zeros((_L, _L), dtype=np.int8)
    np.add.at(counts_t, (idx.reshape(-1), np.repeat(np.arange(_L), _U)), 1)
    return counts_t


_COUNTS_T = _build_counts_t()


def _pair_kernel(qc_ref, kc_ref, qp_ref, kp_ref, vp_ref, c_ref, o_ref,
                 m2_ref, rank_ref, *, bq1, bq2, d):
    L = _L
    scale = 1.0 / math.sqrt(d)
    par = jax.lax.rem(pl.program_id(0), 2)

    j_iota = jax.lax.broadcasted_iota(jnp.int32, (L, bq1), 0)
    l_iota = jax.lax.broadcasted_iota(jnp.int32, (L, bq1), 1)
    d_iota = j_iota - l_iota

    m_rows = []
    m_cols = []
    for hh in range(2):
        m_row = m2_ref[pl.ds(2 * (1 - par) + hh, 1), :]
        m_rows.append(m_row)
        m_cols.append(jnp.transpose(m_row, (1, 0)))

    for i in range(L // bq1):
        c = c_ref[:, pl.ds(i * bq1, bq1)]
        cf = c.astype(jnp.float32)
        for hh in range(2):
            qb = qc_ref[pl.ds(i * bq1, bq1), pl.ds(hh * d, d)]
            kcur = kc_ref[:, pl.ds(hh * d, d)]
            st = jax.lax.dot_general(
                kcur, qb, (((1,), (1,)), ((), ())),
                preferred_element_type=jnp.float32,
            )
            mx = jnp.max(jnp.where(cf > 0.5, st, -3e38), axis=0,
                         keepdims=True)
            sm = jnp.sum(cf * st, axis=0, keepdims=True)
            m2_ref[pl.ds(2 * par + hh, 1), pl.ds(i * bq1, bq1)] = (
                mx - sm * (1.0 / L))

            mb = jax.lax.slice(m_rows[hh], (0, i * bq1), (1, (i + 1) * bq1))
            gt = m_cols[hh] > mb
            tie = (m_cols[hh] == mb) & (d_iota < i * bq1)
            cnt = jnp.sum(jnp.where(gt | tie, 1.0, 0.0), axis=0,
                          keepdims=True)
            rank_ref[pl.ds(hh, 1), pl.ds(i * bq1, bq1)] = cnt

    t_iota = jax.lax.broadcasted_iota(jnp.int32, (_U, L), 0)
    iota_row = jax.lax.broadcasted_iota(jnp.int32, (1, L), 1).astype(
        jnp.float32)
    k_iota = jax.lax.broadcasted_iota(jnp.int32, (L, _U), 0)
    r2 = jax.lax.broadcasted_iota(jnp.int32, (bq2, bq2), 0)
    c2 = jax.lax.broadcasted_iota(jnp.int32, (bq2, bq2), 1)
    tri = (r2 >= c2).astype(jnp.float32)
    nb = L // bq2

    for hh in range(2):
        hs = pl.ds(hh * d, d)
        rank_row = rank_ref[pl.ds(hh, 1), :].astype(jnp.int32)
        p40 = jnp.where(rank_row == t_iota, 1.0, 0.0)

        qsel = jax.lax.dot_general(
            p40, qp_ref[:, hs], (((1,), (0,)), ((), ())),
            preferred_element_type=jnp.float32,
        )
        qidx = jax.lax.dot_general(
            iota_row, p40, (((1,), (1,)), ((), ())),
            preferred_element_type=jnp.float32,
        ).astype(jnp.int32)
        st = jax.lax.dot_general(
            kp_ref[:, hs], qsel, (((1,), (1,)), ((), ())),
            preferred_element_type=jnp.float32,
        ) * scale
        st = jnp.where(k_iota > qidx, _NEG, st)
        st = st - jnp.max(st, axis=0, keepdims=True)
        e = jnp.exp(st)
        p = e / jnp.sum(e, axis=0, keepdims=True)
        osel = jax.lax.dot_general(
            p, vp_ref[:, hs], (((0,), (0,)), ((), ())),
            preferred_element_type=jnp.float32,
        )

        scat = jax.lax.dot_general(
            p40, osel, (((0,), (0,)), ((), ())),
            preferred_element_type=jnp.float32,
        )
        selc = jax.lax.dot_general(
            p40, jnp.ones((_U, 1), jnp.float32), (((0,), (0,)), ((), ())),
            preferred_element_type=jnp.float32,
        )

        vbs = [vp_ref[pl.ds(i * bq2, bq2), hs] for i in range(nb)]
        bsums = [jnp.sum(vb, axis=0, keepdims=True) for vb in vbs]
        carry = jnp.zeros((1, d), jnp.float32)
        for i in range(nb):
            ctx = jax.lax.dot_general(
                tri, vbs[i], (((1,), (0,)), ((), ())),
                preferred_element_type=jnp.float32,
            ) + carry
            selb = jax.lax.slice(selc, (i * bq2, 0), ((i + 1) * bq2, 1))
            scatb = jax.lax.slice(scat, (i * bq2, 0), ((i + 1) * bq2, d))
            o_ref[pl.ds(i * bq2, bq2), hs] = ctx + selb * (scatb - ctx)
            carry = carry + bsums[i]


@jax.jit
def kernel(queries, keys, values):
    B, L, H, D = queries.shape
    HD = H * D
    Q = queries.reshape(B * L, HD)
    K = keys.reshape(B * L, HD)
    V = values.reshape(B * L, HD)
    counts_t = jnp.asarray(_COUNTS_T.astype(jnp.bfloat16))

    bq1, bq2 = 512, 256
    npair = H // 2
    cur = lambda g: (0, jnp.minimum(g, npair - 1))
    prev = lambda g: (0, jnp.maximum(g - 1, 0))
    out = pl.pallas_call(
        functools.partial(_pair_kernel, bq1=bq1, bq2=bq2, d=D),
        grid=(npair + 1,),
        in_specs=[
            pl.BlockSpec((L, 2 * D), cur),
            pl.BlockSpec((L, 2 * D), cur),
            pl.BlockSpec((L, 2 * D), prev),
            pl.BlockSpec((L, 2 * D), prev),
            pl.BlockSpec((L, 2 * D), prev),
            pl.BlockSpec((L, L), lambda g: (0, 0)),
        ],
        out_specs=pl.BlockSpec((L, 2 * D), prev),
        out_shape=jax.ShapeDtypeStruct((B * L, HD), jnp.float32),
        scratch_shapes=[
            pltpu.VMEM((4, L), jnp.float32),
            pltpu.VMEM((2, L), jnp.float32),
        ],
        compiler_params=pltpu.CompilerParams(
            dimension_semantics=("arbitrary",),
            vmem_limit_bytes=100 * 1024 * 1024,
        ),
    )(Q, K, Q, K, V, counts_t)

    return out.reshape(B, L, H, D)
